# Initial kernel scaffold; baseline (speedup 1.0000x reference)
import jax
import jax.numpy as jnp
from jax.experimental import pallas as pl


def kernel(query_embeddings, query_maskings, edge_indexes, edge_weights, edge_maskings, edge_emb_table, ln1_w, ln1_b, Wl, bl, Wr, br, We, att, gat_bias, ffn_W1, ffn_b1, ffn_W2, ffn_b2, norm_w, norm_b):
    raise NotImplementedError("write your pallas kernel here")


# speedup vs baseline: 16.5128x; 16.5128x over previous
import functools

import jax
import jax.numpy as jnp
from jax.experimental import pallas as pl
from jax.experimental.pallas import tpu as pltpu

B, Q, N, E, D, H, ED = 16, 64, 32, 64, 128, 4, 8
G = B * Q
HD = H * D

GB = 16


def _bdot(a, b):
    return jax.lax.dot_general(
        a, b,
        dimension_numbers=(((2,), (1,)), ((0,), (0,))),
        preferred_element_type=jnp.float32)


def _gat_kernel(x_ref, src_ref, dst_ref, w_ref, e3_ref,
                ln1w_ref, ln1b_ref, wl_ref, bl_ref, wr_ref, br_ref,
                att_ref, gbias_ref, w1_ref, b1_ref, w2_ref, b2_ref,
                nw_ref, nb_ref, out_ref):
    x = x_ref[...]
    mu = jnp.mean(x, axis=-1, keepdims=True)
    var = jnp.mean((x - mu) ** 2, axis=-1, keepdims=True)
    xn = (x - mu) * jax.lax.rsqrt(var + 1e-5) * ln1w_ref[...] + ln1b_ref[...]

    xl = jnp.dot(xn, wl_ref[...], preferred_element_type=jnp.float32) + bl_ref[...]
    xr = jnp.dot(xn, wr_ref[...], preferred_element_type=jnp.float32) + br_ref[...]
    xl_g = xl.reshape(GB, N, HD)
    xr_g = xr.reshape(GB, N, HD)

    src = src_ref[...]
    dst = dst_ref[...]
    iota_n = jax.lax.broadcasted_iota(jnp.int32, (GB, E, N), 2)
    oh_src = (src[:, :, None] == iota_n).astype(jnp.float32)
    oh_dst = (dst[:, :, None] == iota_n).astype(jnp.float32)

    xl_src = _bdot(oh_src, xl_g)
    xr_dst = _bdot(oh_dst, xr_g)

    w = w_ref[...].astype(jnp.float32)
    e3 = e3_ref[...]
    em = e3[0][None, None, :] + w[:, :, None] * (e3[1] - e3[0])[None, None, :]

    att = att_ref[...].reshape(1, HD)

    m = xl_src + xr_dst + em
    m = jnp.where(m > 0, m, 0.2 * m)
    lg = (m * att[None]).reshape(GB * E * H, D)
    logits_e = jnp.sum(lg, axis=-1).reshape(GB, E, H)

    ms = xl_g + xr_g + e3[2][None, None, :]
    ms = jnp.where(ms > 0, ms, 0.2 * ms)
    lgs = (ms * att[None]).reshape(GB * N * H, D)
    logits_s = jnp.sum(lgs, axis=-1).reshape(GB, N, H)

    mx_cols = []
    for h in range(H):
        lh = logits_e[:, :, h]
        masked = jnp.where(oh_dst > 0, lh[:, :, None], -1e30)
        mx_cols.append(jnp.max(masked, axis=1))
    mx = jnp.stack(mx_cols, axis=-1)
    mx = jnp.maximum(mx, logits_s)

    mx_dst = _bdot(oh_dst, mx)
    aexp_e = jnp.exp(logits_e - mx_dst)
    aexp_s = jnp.exp(logits_s - mx)

    den = _bdot(oh_dst.transpose(0, 2, 1), aexp_e) + aexp_s
    den_dst = _bdot(oh_dst, den)
    alpha_e = aexp_e / (den_dst + 1e-16)
    alpha_s = aexp_s / (den + 1e-16)

    msg = (xl_src.reshape(GB, E, H, D) * alpha_e[:, :, :, None]).reshape(GB, E, HD)
    out = _bdot(oh_dst.transpose(0, 2, 1), msg)
    out = out + (xl_g.reshape(GB, N, H, D) * alpha_s[:, :, :, None]).reshape(GB, N, HD)

    o = jnp.sum(out.reshape(GB * N, H, D), axis=1) * (1.0 / H) + gbias_ref[...]

    h1 = jnp.dot(o, w1_ref[...], preferred_element_type=jnp.float32) + b1_ref[...]
    h1 = jnp.where(h1 > 0, h1, 0.01 * h1)
    h2 = jnp.dot(h1, w2_ref[...], preferred_element_type=jnp.float32) + b2_ref[...]

    gf = jnp.sum(h2.reshape(GB, N, D), axis=1) * (1.0 / N)
    mu2 = jnp.mean(gf, axis=-1, keepdims=True)
    var2 = jnp.mean((gf - mu2) ** 2, axis=-1, keepdims=True)
    out_ref[...] = (gf - mu2) * jax.lax.rsqrt(var2 + 1e-5) * nw_ref[...] + nb_ref[...]


@jax.jit
def _run(x_nodes, src, dst, ew, e3, ln1_w, ln1_b, WlT, bl, WrT, br,
         att, gat_bias, W1T, b1, W2T, b2, norm_w, norm_b):
    grid = (G // GB,)
    full = lambda shape: pl.BlockSpec(shape, lambda i: (0,) * len(shape))
    out = pl.pallas_call(
        _gat_kernel,
        grid=grid,
        in_specs=[
            pl.BlockSpec((GB * N, D), lambda i: (i, 0)),
            pl.BlockSpec((GB, E), lambda i: (i, 0)),
            pl.BlockSpec((GB, E), lambda i: (i, 0)),
            pl.BlockSpec((GB, E), lambda i: (i, 0)),
            full((8, HD)),
            full((1, D)), full((1, D)),
            full((D, HD)), full((1, HD)),
            full((D, HD)), full((1, HD)),
            full((H, D)),
            full((1, D)),
            full((D, D)), full((1, D)),
            full((D, D)), full((1, D)),
            full((1, D)), full((1, D)),
        ],
        out_specs=pl.BlockSpec((GB, D), lambda i: (i, 0)),
        out_shape=jax.ShapeDtypeStruct((G, D), jnp.float32),
    )(x_nodes, src, dst, ew, e3, ln1_w, ln1_b, WlT, bl, WrT, br,
      att, gat_bias, W1T, b1, W2T, b2, norm_w, norm_b)
    return out.reshape(B, Q, D)


def kernel(query_embeddings, query_maskings, edge_indexes, edge_weights,
           edge_maskings, edge_emb_table, ln1_w, ln1_b, Wl, bl, Wr, br, We,
           att, gat_bias, ffn_W1, ffn_b1, ffn_W2, ffn_b2, norm_w, norm_b):
    x_nodes = query_embeddings.reshape(G * N, D)
    eidx = edge_indexes.reshape(G, 2, E)
    src = eidx[:, 0, :]
    dst = eidx[:, 1, :]
    ew = edge_weights.reshape(G, E)

    p1 = jnp.mean(ew.astype(jnp.float32))
    ea_mean = edge_emb_table[0] + p1 * (edge_emb_table[1] - edge_emb_table[0])
    e3 = jnp.concatenate(
        [edge_emb_table, ea_mean[None, :],
         jnp.zeros((5, ED), jnp.float32)], axis=0) @ We.T

    return _run(
        x_nodes, src, dst, ew, e3,
        ln1_w.reshape(1, D), ln1_b.reshape(1, D),
        Wl.T, bl.reshape(1, HD), Wr.T, br.reshape(1, HD),
        att, gat_bias.reshape(1, D),
        ffn_W1.T, ffn_b1.reshape(1, D), ffn_W2.T, ffn_b2.reshape(1, D),
        norm_w.reshape(1, D), norm_b.reshape(1, D))


# speedup vs baseline: 42.0925x; 2.5491x over previous
import functools

import jax
import jax.numpy as jnp
from jax.experimental import pallas as pl
from jax.experimental.pallas import tpu as pltpu

B, Q, N, E, D, H, ED = 16, 64, 32, 64, 128, 4, 8
G = B * Q
HD = H * D

GB = 16


def _bdot(a, b):
    return jax.lax.dot_general(
        a, b,
        dimension_numbers=(((2,), (1,)), ((0,), (0,))),
        preferred_element_type=jnp.float32)


def _gat_kernel(x_ref, src_ref, dst_ref, w_ref, e3_ref,
                ln1w_ref, ln1b_ref, wl_ref, bl_ref, wr_ref, br_ref,
                attbd_ref, onesbd_ref, meanbd_ref,
                gbias_ref, w1_ref, b1_ref, w2_ref, b2_ref,
                nw_ref, nb_ref, out_ref):
    x = x_ref[...]
    mu = jnp.mean(x, axis=-1, keepdims=True)
    var = jnp.mean((x - mu) ** 2, axis=-1, keepdims=True)
    xn = (x - mu) * jax.lax.rsqrt(var + 1e-5) * ln1w_ref[...] + ln1b_ref[...]

    xl = jnp.dot(xn, wl_ref[...], preferred_element_type=jnp.float32) + bl_ref[...]
    xr = jnp.dot(xn, wr_ref[...], preferred_element_type=jnp.float32) + br_ref[...]
    xl_g = xl.reshape(GB, N, HD)
    xr_g = xr.reshape(GB, N, HD)

    src = src_ref[...]
    dst = dst_ref[...]
    iota_n = jax.lax.broadcasted_iota(jnp.int32, (GB, E, N), 2)
    oh_src = (src[:, :, None] == iota_n).astype(jnp.float32)
    oh_dst = (dst[:, :, None] == iota_n).astype(jnp.float32)

    xl_src = _bdot(oh_src, xl_g)
    xr_dst = _bdot(oh_dst, xr_g)

    w = w_ref[...].astype(jnp.float32)
    e3 = e3_ref[...]
    em = (e3[0][None, None, :] +
          w[:, :, None] * (e3[1] - e3[0])[None, None, :]).reshape(GB * E, HD)

    att_bd = attbd_ref[...]
    ones_bd = onesbd_ref[...]

    m = xl_src.reshape(GB * E, HD) + xr_dst.reshape(GB * E, HD) + em
    m = jnp.where(m > 0, m, 0.2 * m)
    logits_e = jnp.dot(m, att_bd,
                       preferred_element_type=jnp.float32).reshape(GB, E, 8)

    ms = (xl + xr).reshape(GB * N, HD) + e3[2][None, :]
    ms = jnp.where(ms > 0, ms, 0.2 * ms)
    logits_s = jnp.dot(ms, att_bd,
                       preferred_element_type=jnp.float32).reshape(GB, N, 8)

    mx_cols = []
    for h in range(H):
        lh = logits_e[:, :, h]
        masked = jnp.where(oh_dst > 0, lh[:, :, None], -1e30)
        mx_cols.append(jnp.max(masked, axis=1))
    z = jnp.zeros((GB, N), jnp.float32)
    mx = jnp.stack(mx_cols + [z, z, z, z], axis=-1)
    mx = jnp.maximum(mx, logits_s)

    mx_dst = _bdot(oh_dst, mx)
    aexp_e = jnp.exp(logits_e - mx_dst)
    aexp_s = jnp.exp(logits_s - mx)

    den = _bdot(oh_dst.transpose(0, 2, 1), aexp_e) + aexp_s
    den_dst = _bdot(oh_dst, den)
    alpha_e = aexp_e / (den_dst + 1e-16)
    alpha_s = aexp_s / (den + 1e-16)

    alpha_bc = jnp.dot(alpha_e.reshape(GB * E, 8), ones_bd,
                       preferred_element_type=jnp.float32)
    msg = (xl_src.reshape(GB * E, HD) * alpha_bc).reshape(GB, E, HD)
    out = _bdot(oh_dst.transpose(0, 2, 1), msg)
    alpha_s_bc = jnp.dot(alpha_s.reshape(GB * N, 8), ones_bd,
                         preferred_element_type=jnp.float32)
    out = out.reshape(GB * N, HD) + xl * alpha_s_bc

    o = jnp.dot(out, meanbd_ref[...], preferred_element_type=jnp.float32) + gbias_ref[...]

    h1 = jnp.dot(o, w1_ref[...], preferred_element_type=jnp.float32) + b1_ref[...]
    h1 = jnp.where(h1 > 0, h1, 0.01 * h1)
    h2 = jnp.dot(h1, w2_ref[...], preferred_element_type=jnp.float32) + b2_ref[...]

    gf = jnp.sum(h2.reshape(GB, N, D), axis=1) * (1.0 / N)
    mu2 = jnp.mean(gf, axis=-1, keepdims=True)
    var2 = jnp.mean((gf - mu2) ** 2, axis=-1, keepdims=True)
    out_ref[...] = (gf - mu2) * jax.lax.rsqrt(var2 + 1e-5) * nw_ref[...] + nb_ref[...]


@jax.jit
def _run(x_nodes, src, dst, ew, e3, ln1_w, ln1_b, WlT, bl, WrT, br,
         att_bd, ones_bd, mean_bd, gat_bias, W1T, b1, W2T, b2, norm_w, norm_b):
    grid = (G // GB,)
    full = lambda shape: pl.BlockSpec(shape, lambda i: (0,) * len(shape))
    out = pl.pallas_call(
        _gat_kernel,
        grid=grid,
        in_specs=[
            pl.BlockSpec((GB * N, D), lambda i: (i, 0)),
            pl.BlockSpec((GB, E), lambda i: (i, 0)),
            pl.BlockSpec((GB, E), lambda i: (i, 0)),
            pl.BlockSpec((GB, E), lambda i: (i, 0)),
            full((8, HD)),
            full((1, D)), full((1, D)),
            full((D, HD)), full((1, HD)),
            full((D, HD)), full((1, HD)),
            full((HD, 8)),
            full((8, HD)),
            full((HD, D)),
            full((1, D)),
            full((D, D)), full((1, D)),
            full((D, D)), full((1, D)),
            full((1, D)), full((1, D)),
        ],
        out_specs=pl.BlockSpec((GB, D), lambda i: (i, 0)),
        out_shape=jax.ShapeDtypeStruct((G, D), jnp.float32),
    )(x_nodes, src, dst, ew, e3, ln1_w, ln1_b, WlT, bl, WrT, br,
      att_bd, ones_bd, mean_bd, gat_bias, W1T, b1, W2T, b2, norm_w, norm_b)
    return out.reshape(B, Q, D)


def kernel(query_embeddings, query_maskings, edge_indexes, edge_weights,
           edge_maskings, edge_emb_table, ln1_w, ln1_b, Wl, bl, Wr, br, We,
           att, gat_bias, ffn_W1, ffn_b1, ffn_W2, ffn_b2, norm_w, norm_b):
    x_nodes = query_embeddings.reshape(G * N, D)
    eidx = edge_indexes.reshape(G, 2, E)
    src = eidx[:, 0, :]
    dst = eidx[:, 1, :]
    ew = edge_weights.reshape(G, E)

    p1 = jnp.mean(ew.astype(jnp.float32))
    ea_mean = edge_emb_table[0] + p1 * (edge_emb_table[1] - edge_emb_table[0])
    e3 = jnp.concatenate(
        [edge_emb_table, ea_mean[None, :],
         jnp.zeros((5, ED), jnp.float32)], axis=0) @ We.T

    hid = jnp.arange(HD, dtype=jnp.int32)
    h8 = jnp.arange(8, dtype=jnp.int32)
    att_bd = jnp.where((hid[:, None] // D) == h8[None, :],
                       att.reshape(-1)[:, None], 0.0)
    ones_bd = ((h8[:, None] < H) & (hid[None, :] // D == h8[:, None])
               ).astype(jnp.float32)
    mean_bd = ((hid[:, None] % D) == jnp.arange(D)[None, :]).astype(
        jnp.float32) * (1.0 / H)

    return _run(
        x_nodes, src, dst, ew, e3,
        ln1_w.reshape(1, D), ln1_b.reshape(1, D),
        Wl.T, bl.reshape(1, HD), Wr.T, br.reshape(1, HD),
        att_bd, ones_bd, mean_bd, gat_bias.reshape(1, D),
        ffn_W1.T, ffn_b1.reshape(1, D), ffn_W2.T, ffn_b2.reshape(1, D),
        norm_w.reshape(1, D), norm_b.reshape(1, D))


# speedup vs baseline: 48.9194x; 1.1622x over previous
import functools

import jax
import jax.numpy as jnp
from jax.experimental import pallas as pl
from jax.experimental.pallas import tpu as pltpu

B, Q, N, E, D, H, ED = 16, 64, 32, 64, 128, 4, 8
G = B * Q
HD = H * D

GB = 16


def _bdot(a, b):
    return jax.lax.dot_general(
        a, b,
        dimension_numbers=(((2,), (1,)), ((0,), (0,))),
        preferred_element_type=jnp.float32)


def _bdot16(a, b):
    return jax.lax.dot_general(
        a.astype(jnp.bfloat16), b.astype(jnp.bfloat16),
        dimension_numbers=(((2,), (1,)), ((0,), (0,))),
        preferred_element_type=jnp.float32)


def _dot16(a, b):
    return jnp.dot(a.astype(jnp.bfloat16), b.astype(jnp.bfloat16),
                   preferred_element_type=jnp.float32)


def _gat_kernel(x_ref, src_ref, dst_ref, w_ref, e3_ref,
                ln1w_ref, ln1b_ref, wl_ref, bl_ref, wr_ref, br_ref,
                attbd_ref, onesbd_ref, meanbd_ref,
                gbias_ref, w1_ref, b1_ref, w2_ref, b2_ref,
                nw_ref, nb_ref, out_ref):
    x = x_ref[...]
    mu = jnp.mean(x, axis=-1, keepdims=True)
    var = jnp.mean((x - mu) ** 2, axis=-1, keepdims=True)
    xn = (x - mu) * jax.lax.rsqrt(var + 1e-5) * ln1w_ref[...] + ln1b_ref[...]

    xl = _dot16(xn, wl_ref[...]) + bl_ref[...]
    xr = _dot16(xn, wr_ref[...]) + br_ref[...]
    xl_g = xl.reshape(GB, N, HD)
    xr_g = xr.reshape(GB, N, HD)

    src = src_ref[...]
    dst = dst_ref[...]
    iota_n = jax.lax.broadcasted_iota(jnp.int32, (GB, E, N), 2)
    oh_src = (src[:, :, None] == iota_n).astype(jnp.float32)
    oh_dst = (dst[:, :, None] == iota_n).astype(jnp.float32)

    xl_src = _bdot16(oh_src, xl_g)
    xr_dst = _bdot16(oh_dst, xr_g)

    w = w_ref[...].astype(jnp.float32)
    e3 = e3_ref[...]
    em = (e3[0][None, None, :] +
          w[:, :, None] * (e3[1] - e3[0])[None, None, :]).reshape(GB * E, HD)

    att_bd = attbd_ref[...]
    ones_bd = onesbd_ref[...]

    m = xl_src.reshape(GB * E, HD) + xr_dst.reshape(GB * E, HD) + em
    m = jnp.where(m > 0, m, 0.2 * m)
    logits_e = jnp.dot(m, att_bd,
                       preferred_element_type=jnp.float32).reshape(GB, E, 8)

    ms = (xl + xr).reshape(GB * N, HD) + e3[2][None, :]
    ms = jnp.where(ms > 0, ms, 0.2 * ms)
    logits_s = jnp.dot(ms, att_bd,
                       preferred_element_type=jnp.float32).reshape(GB, N, 8)

    mx_dst = _bdot(oh_dst, logits_s)
    aexp_e = jnp.exp(logits_e - mx_dst)

    den = _bdot(oh_dst.transpose(0, 2, 1), aexp_e) + 1.0
    den_dst = _bdot(oh_dst, den)
    alpha_e = aexp_e / (den_dst + 1e-16)
    alpha_s = 1.0 / (den + 1e-16)

    alpha_bc = _dot16(alpha_e.reshape(GB * E, 8), ones_bd)
    msg = (xl_src.reshape(GB * E, HD) * alpha_bc).reshape(GB, E, HD)
    out = _bdot16(oh_dst.transpose(0, 2, 1), msg)
    alpha_s_bc = _dot16(alpha_s.reshape(GB * N, 8), ones_bd)
    out = out.reshape(GB * N, HD) + xl * alpha_s_bc

    o = _dot16(out, meanbd_ref[...]) + gbias_ref[...]

    h1 = _dot16(o, w1_ref[...]) + b1_ref[...]
    h1 = jnp.where(h1 > 0, h1, 0.01 * h1)
    h2 = _dot16(h1, w2_ref[...]) + b2_ref[...]

    gf = jnp.sum(h2.reshape(GB, N, D), axis=1) * (1.0 / N)
    mu2 = jnp.mean(gf, axis=-1, keepdims=True)
    var2 = jnp.mean((gf - mu2) ** 2, axis=-1, keepdims=True)
    out_ref[...] = (gf - mu2) * jax.lax.rsqrt(var2 + 1e-5) * nw_ref[...] + nb_ref[...]


@jax.jit
def _run(x_nodes, src, dst, ew, e3, ln1_w, ln1_b, WlT, bl, WrT, br,
         att_bd, ones_bd, mean_bd, gat_bias, W1T, b1, W2T, b2, norm_w, norm_b):
    grid = (G // GB,)
    full = lambda shape: pl.BlockSpec(shape, lambda i: (0,) * len(shape))
    out = pl.pallas_call(
        _gat_kernel,
        grid=grid,
        in_specs=[
            pl.BlockSpec((GB * N, D), lambda i: (i, 0)),
            pl.BlockSpec((GB, E), lambda i: (i, 0)),
            pl.BlockSpec((GB, E), lambda i: (i, 0)),
            pl.BlockSpec((GB, E), lambda i: (i, 0)),
            full((8, HD)),
            full((1, D)), full((1, D)),
            full((D, HD)), full((1, HD)),
            full((D, HD)), full((1, HD)),
            full((HD, 8)),
            full((8, HD)),
            full((HD, D)),
            full((1, D)),
            full((D, D)), full((1, D)),
            full((D, D)), full((1, D)),
            full((1, D)), full((1, D)),
        ],
        out_specs=pl.BlockSpec((GB, D), lambda i: (i, 0)),
        out_shape=jax.ShapeDtypeStruct((G, D), jnp.float32),
    )(x_nodes, src, dst, ew, e3, ln1_w, ln1_b, WlT, bl, WrT, br,
      att_bd, ones_bd, mean_bd, gat_bias, W1T, b1, W2T, b2, norm_w, norm_b)
    return out.reshape(B, Q, D)


def kernel(query_embeddings, query_maskings, edge_indexes, edge_weights,
           edge_maskings, edge_emb_table, ln1_w, ln1_b, Wl, bl, Wr, br, We,
           att, gat_bias, ffn_W1, ffn_b1, ffn_W2, ffn_b2, norm_w, norm_b):
    x_nodes = query_embeddings.reshape(G * N, D)
    eidx = edge_indexes.reshape(G, 2, E)
    src = eidx[:, 0, :]
    dst = eidx[:, 1, :]
    ew = edge_weights.reshape(G, E)

    p1 = jnp.mean(ew.astype(jnp.float32))
    ea_mean = edge_emb_table[0] + p1 * (edge_emb_table[1] - edge_emb_table[0])
    e3 = jnp.concatenate(
        [edge_emb_table, ea_mean[None, :],
         jnp.zeros((5, ED), jnp.float32)], axis=0) @ We.T

    hid = jnp.arange(HD, dtype=jnp.int32)
    h8 = jnp.arange(8, dtype=jnp.int32)
    att_bd = jnp.where((hid[:, None] // D) == h8[None, :],
                       att.reshape(-1)[:, None], 0.0)
    ones_bd = ((h8[:, None] < H) & (hid[None, :] // D == h8[:, None])
               ).astype(jnp.float32)
    mean_bd = ((hid[:, None] % D) == jnp.arange(D)[None, :]).astype(
        jnp.float32) * (1.0 / H)

    return _run(
        x_nodes, src, dst, ew, e3,
        ln1_w.reshape(1, D), ln1_b.reshape(1, D),
        Wl.T, bl.reshape(1, HD), Wr.T, br.reshape(1, HD),
        att_bd, ones_bd, mean_bd, gat_bias.reshape(1, D),
        ffn_W1.T, ffn_b1.reshape(1, D), ffn_W2.T, ffn_b2.reshape(1, D),
        norm_w.reshape(1, D), norm_b.reshape(1, D))


# speedup vs baseline: 53.2867x; 1.0893x over previous
import functools

import jax
import jax.numpy as jnp
from jax.experimental import pallas as pl
from jax.experimental.pallas import tpu as pltpu

B, Q, N, E, D, H, ED = 16, 64, 32, 64, 128, 4, 8
G = B * Q
HD = H * D

GB = 16


def _bdot(a, b):
    return jax.lax.dot_general(
        a, b,
        dimension_numbers=(((2,), (1,)), ((0,), (0,))),
        preferred_element_type=jnp.float32)


def _bdot16(a, b, out_dtype=jnp.float32):
    return jax.lax.dot_general(
        a.astype(jnp.bfloat16), b.astype(jnp.bfloat16),
        dimension_numbers=(((2,), (1,)), ((0,), (0,))),
        preferred_element_type=out_dtype)


def _dot16(a, b, out_dtype=jnp.float32):
    return jnp.dot(a.astype(jnp.bfloat16), b.astype(jnp.bfloat16),
                   preferred_element_type=out_dtype)


def _gat_kernel(x_ref, src_ref, dst_ref, w_ref, e3_ref,
                ln1w_ref, ln1b_ref, wl_ref, bl_ref, wr_ref, br_ref,
                attbd_ref, onesbd_ref, meanbd_ref,
                gbias_ref, w1_ref, b1_ref, w2_ref, b2_ref,
                nw_ref, nb_ref, out_ref):
    x = x_ref[...]
    mu = jnp.mean(x, axis=-1, keepdims=True)
    var = jnp.mean((x - mu) ** 2, axis=-1, keepdims=True)
    xn = (x - mu) * jax.lax.rsqrt(var + 1e-5) * ln1w_ref[...] + ln1b_ref[...]

    bf16 = jnp.bfloat16

    xl = (_dot16(xn, wl_ref[...]) + bl_ref[...]).astype(bf16)
    xr = (_dot16(xn, wr_ref[...]) + br_ref[...]).astype(bf16)
    xl_g = xl.reshape(GB, N, HD)
    xr_g = xr.reshape(GB, N, HD)

    src = src_ref[...]
    dst = dst_ref[...]
    iota_n = jax.lax.broadcasted_iota(jnp.int32, (GB, E, N), 2)
    oh_src = (src[:, :, None] == iota_n).astype(bf16)
    oh_dst = (dst[:, :, None] == iota_n).astype(bf16)

    xl_src = _bdot16(oh_src, xl_g).astype(bf16).reshape(GB * E, HD)
    xr_dst = _bdot16(oh_dst, xr_g).astype(bf16).reshape(GB * E, HD)

    w = w_ref[...].astype(bf16)
    e3 = e3_ref[...].astype(bf16)
    em = (e3[0][None, None, :] +
          w[:, :, None] * (e3[1] - e3[0])[None, None, :]).reshape(GB * E, HD)

    att_bd = attbd_ref[...].astype(bf16)
    ones_bd = onesbd_ref[...].astype(bf16)

    m = xl_src + xr_dst + em
    m = jnp.where(m > 0, m, 0.2 * m)
    logits_e = jnp.dot(m, att_bd,
                       preferred_element_type=jnp.float32).reshape(GB, E, 8)

    ms = (xl + xr) + e3[2][None, :]
    ms = jnp.where(ms > 0, ms, 0.2 * ms)
    logits_s = jnp.dot(ms, att_bd,
                       preferred_element_type=jnp.float32).reshape(GB, N, 8)

    mx_dst = _bdot16(oh_dst, logits_s)
    aexp_e = jnp.exp(logits_e - mx_dst)

    den = _bdot16(oh_dst.transpose(0, 2, 1), aexp_e) + 1.0
    den_dst = _bdot16(oh_dst, den)
    alpha_e = aexp_e / (den_dst + 1e-16)
    alpha_s = 1.0 / (den + 1e-16)

    alpha_bc = _dot16(alpha_e.reshape(GB * E, 8), ones_bd).astype(bf16)
    msg = (xl_src * alpha_bc).reshape(GB, E, HD)
    out = _bdot16(oh_dst.transpose(0, 2, 1), msg).astype(bf16)
    alpha_s_bc = _dot16(alpha_s.reshape(GB * N, 8), ones_bd).astype(bf16)
    out = out.reshape(GB * N, HD) + xl * alpha_s_bc

    o = _dot16(out, meanbd_ref[...]) + gbias_ref[...]

    h1 = _dot16(o, w1_ref[...]) + b1_ref[...]
    h1 = jnp.where(h1 > 0, h1, 0.01 * h1)
    h2 = _dot16(h1, w2_ref[...]) + b2_ref[...]

    gf = jnp.sum(h2.reshape(GB, N, D), axis=1) * (1.0 / N)
    mu2 = jnp.mean(gf, axis=-1, keepdims=True)
    var2 = jnp.mean((gf - mu2) ** 2, axis=-1, keepdims=True)
    out_ref[...] = (gf - mu2) * jax.lax.rsqrt(var2 + 1e-5) * nw_ref[...] + nb_ref[...]


@jax.jit
def _run(x_nodes, src, dst, ew, e3, ln1_w, ln1_b, WlT, bl, WrT, br,
         att_bd, ones_bd, mean_bd, gat_bias, W1T, b1, W2T, b2, norm_w, norm_b):
    grid = (G // GB,)
    full = lambda shape: pl.BlockSpec(shape, lambda i: (0,) * len(shape))
    out = pl.pallas_call(
        _gat_kernel,
        grid=grid,
        in_specs=[
            pl.BlockSpec((GB * N, D), lambda i: (i, 0)),
            pl.BlockSpec((GB, E), lambda i: (i, 0)),
            pl.BlockSpec((GB, E), lambda i: (i, 0)),
            pl.BlockSpec((GB, E), lambda i: (i, 0)),
            full((8, HD)),
            full((1, D)), full((1, D)),
            full((D, HD)), full((1, HD)),
            full((D, HD)), full((1, HD)),
            full((HD, 8)),
            full((8, HD)),
            full((HD, D)),
            full((1, D)),
            full((D, D)), full((1, D)),
            full((D, D)), full((1, D)),
            full((1, D)), full((1, D)),
        ],
        out_specs=pl.BlockSpec((GB, D), lambda i: (i, 0)),
        out_shape=jax.ShapeDtypeStruct((G, D), jnp.float32),
    )(x_nodes, src, dst, ew, e3, ln1_w, ln1_b, WlT, bl, WrT, br,
      att_bd, ones_bd, mean_bd, gat_bias, W1T, b1, W2T, b2, norm_w, norm_b)
    return out.reshape(B, Q, D)


def kernel(query_embeddings, query_maskings, edge_indexes, edge_weights,
           edge_maskings, edge_emb_table, ln1_w, ln1_b, Wl, bl, Wr, br, We,
           att, gat_bias, ffn_W1, ffn_b1, ffn_W2, ffn_b2, norm_w, norm_b):
    x_nodes = query_embeddings.reshape(G * N, D)
    eidx = edge_indexes.reshape(G, 2, E)
    src = eidx[:, 0, :]
    dst = eidx[:, 1, :]
    ew = edge_weights.reshape(G, E)

    p1 = jnp.mean(ew.astype(jnp.float32))
    ea_mean = edge_emb_table[0] + p1 * (edge_emb_table[1] - edge_emb_table[0])
    e3 = jnp.concatenate(
        [edge_emb_table, ea_mean[None, :],
         jnp.zeros((5, ED), jnp.float32)], axis=0) @ We.T

    hid = jnp.arange(HD, dtype=jnp.int32)
    h8 = jnp.arange(8, dtype=jnp.int32)
    att_bd = jnp.where((hid[:, None] // D) == h8[None, :],
                       att.reshape(-1)[:, None], 0.0)
    ones_bd = ((h8[:, None] < H) & (hid[None, :] // D == h8[:, None])
               ).astype(jnp.float32)
    mean_bd = ((hid[:, None] % D) == jnp.arange(D)[None, :]).astype(
        jnp.float32) * (1.0 / H)

    return _run(
        x_nodes, src, dst, ew, e3,
        ln1_w.reshape(1, D), ln1_b.reshape(1, D),
        Wl.T, bl.reshape(1, HD), Wr.T, br.reshape(1, HD),
        att_bd, ones_bd, mean_bd, gat_bias.reshape(1, D),
        ffn_W1.T, ffn_b1.reshape(1, D), ffn_W2.T, ffn_b2.reshape(1, D),
        norm_w.reshape(1, D), norm_b.reshape(1, D))


# speedup vs baseline: 63.9265x; 1.1997x over previous
import functools

import jax
import jax.numpy as jnp
from jax.experimental import pallas as pl
from jax.experimental.pallas import tpu as pltpu

B, Q, N, E, D, H, ED = 16, 64, 32, 64, 128, 4, 8
G = B * Q
HD = H * D

GB = 16


def _bdot(a, b):
    return jax.lax.dot_general(
        a, b,
        dimension_numbers=(((2,), (1,)), ((0,), (0,))),
        preferred_element_type=jnp.float32)


def _bdot16(a, b, out_dtype=jnp.float32):
    return jax.lax.dot_general(
        a.astype(jnp.bfloat16), b.astype(jnp.bfloat16),
        dimension_numbers=(((2,), (1,)), ((0,), (0,))),
        preferred_element_type=out_dtype)


def _dot16(a, b, out_dtype=jnp.float32):
    return jnp.dot(a.astype(jnp.bfloat16), b.astype(jnp.bfloat16),
                   preferred_element_type=out_dtype)


def _gat_kernel(x_ref, src_ref, dst_ref, w_ref, e3_ref,
                ln1w_ref, ln1b_ref, wl_ref, bl_ref, wr_ref, br_ref,
                attbd_ref, onesbd_ref,
                gbias_ref, w1_ref, b1_ref, w2_ref, b2_ref,
                nw_ref, nb_ref, out_ref, xls_ref):
    x = x_ref[...]
    mu = jnp.mean(x, axis=-1, keepdims=True)
    var = jnp.mean((x - mu) ** 2, axis=-1, keepdims=True)
    xn = (x - mu) * jax.lax.rsqrt(var + 1e-5) * ln1w_ref[...] + ln1b_ref[...]

    bf16 = jnp.bfloat16

    xl = (_dot16(xn, wl_ref[...]) + bl_ref[...]).astype(bf16)
    xr = (_dot16(xn, wr_ref[...]) + br_ref[...]).astype(bf16)
    xl_g = xl.reshape(GB, N, HD)
    xr_g = xr.reshape(GB, N, HD)

    src = src_ref[...]
    dst = dst_ref[...]
    iota_n = jax.lax.broadcasted_iota(jnp.int32, (GB, E, N), 2)
    oh_src = (src[:, :, None] == iota_n).astype(bf16)
    oh_dst = (dst[:, :, None] == iota_n).astype(bf16)
    iota_8 = jax.lax.broadcasted_iota(jnp.int32, (GB, E, 8), 2)
    w_oh = (w_ref[...][:, :, None] == iota_8).astype(bf16)

    e3 = e3_ref[...].astype(bf16)
    att_bd = attbd_ref[...].astype(bf16)
    ones_bd32 = onesbd_ref[...].astype(bf16)

    lhs_cat = jnp.concatenate([oh_src, oh_dst, w_oh], axis=-1)
    e3_b = jnp.broadcast_to(e3[None], (GB, 8, HD))
    rhs_cat = jnp.concatenate([xl_g, xr_g, e3_b], axis=1)
    m = _bdot16(lhs_cat, rhs_cat).astype(bf16).reshape(GB * E, HD)

    m = jnp.where(m > 0, m, 0.2 * m)
    logits_e = jnp.dot(m, att_bd,
                       preferred_element_type=jnp.float32).reshape(GB, E, 8)

    ms = (xl + xr) + e3[2][None, :]
    ms = jnp.where(ms > 0, ms, 0.2 * ms)
    logits_s = jnp.dot(ms, att_bd,
                       preferred_element_type=jnp.float32).reshape(GB, N, 8)

    mx_dst = _bdot16(oh_dst, logits_s)
    aexp_e = jnp.exp(logits_e - mx_dst)

    den = _bdot16(oh_dst.transpose(0, 2, 1), aexp_e) + 1.0
    den_dst = _bdot16(oh_dst, den)
    alpha_e = aexp_e / (den_dst + 1e-16)
    alpha_s = 1.0 / (den + 1e-16)

    alpha_lane = _dot16(alpha_e.reshape(GB * E, 8), ones_bd32).astype(bf16)
    oh_src4 = jnp.concatenate([oh_src] * H, axis=-1)
    rhs_a = alpha_lane.reshape(GB, E, H * N) * oh_src4
    a_cat = _bdot16(oh_dst.transpose(0, 2, 1), rhs_a)
    alpha_s_lane = _dot16(alpha_s.reshape(GB * N, 8), ones_bd32)
    r_i = jax.lax.broadcasted_iota(jnp.int32, (N, H * N), 0)
    c_i = jax.lax.broadcasted_iota(jnp.int32, (N, H * N), 1)
    eye4 = (r_i == (c_i % N)).astype(jnp.float32)
    a_cat = (a_cat + eye4[None] * alpha_s_lane.reshape(GB, N, H * N)).astype(bf16)

    for h in range(H):
        xls_ref[:, h * N:(h + 1) * N, :] = xl_g[:, :, h * D:(h + 1) * D]
    out = _bdot16(a_cat, xls_ref[...])
    o = out.reshape(GB * N, D) + gbias_ref[...]

    h1 = _dot16(o, w1_ref[...]) + b1_ref[...]
    h1 = jnp.where(h1 > 0, h1, 0.01 * h1)
    h2 = _dot16(h1, w2_ref[...]) + b2_ref[...]

    gf = jnp.sum(h2.reshape(GB, N, D), axis=1) * (1.0 / N)
    mu2 = jnp.mean(gf, axis=-1, keepdims=True)
    var2 = jnp.mean((gf - mu2) ** 2, axis=-1, keepdims=True)
    out_ref[...] = (gf - mu2) * jax.lax.rsqrt(var2 + 1e-5) * nw_ref[...] + nb_ref[...]


@jax.jit
def _run(x_nodes, src, dst, ew, e3, ln1_w, ln1_b, WlT, bl, WrT, br,
         att_bd, ones_bd32, gat_bias, W1T, b1, W2T, b2, norm_w, norm_b):
    grid = (G // GB,)
    full = lambda shape: pl.BlockSpec(shape, lambda i: (0,) * len(shape))
    out = pl.pallas_call(
        _gat_kernel,
        grid=grid,
        in_specs=[
            pl.BlockSpec((GB * N, D), lambda i: (i, 0)),
            pl.BlockSpec((GB, E), lambda i: (i, 0)),
            pl.BlockSpec((GB, E), lambda i: (i, 0)),
            pl.BlockSpec((GB, E), lambda i: (i, 0)),
            full((8, HD)),
            full((1, D)), full((1, D)),
            full((D, HD)), full((1, HD)),
            full((D, HD)), full((1, HD)),
            full((HD, 8)),
            full((8, H * N)),
            full((1, D)),
            full((D, D)), full((1, D)),
            full((D, D)), full((1, D)),
            full((1, D)), full((1, D)),
        ],
        out_specs=pl.BlockSpec((GB, D), lambda i: (i, 0)),
        out_shape=jax.ShapeDtypeStruct((G, D), jnp.float32),
        scratch_shapes=[pltpu.VMEM((GB, H * N, D), jnp.bfloat16)],
    )(x_nodes, src, dst, ew, e3, ln1_w, ln1_b, WlT, bl, WrT, br,
      att_bd, ones_bd32, gat_bias, W1T, b1, W2T, b2, norm_w, norm_b)
    return out.reshape(B, Q, D)


def kernel(query_embeddings, query_maskings, edge_indexes, edge_weights,
           edge_maskings, edge_emb_table, ln1_w, ln1_b, Wl, bl, Wr, br, We,
           att, gat_bias, ffn_W1, ffn_b1, ffn_W2, ffn_b2, norm_w, norm_b):
    x_nodes = query_embeddings.reshape(G * N, D)
    eidx = edge_indexes.reshape(G, 2, E)
    src = eidx[:, 0, :]
    dst = eidx[:, 1, :]
    ew = edge_weights.reshape(G, E)

    p1 = jnp.mean(ew.astype(jnp.float32))
    ea_mean = edge_emb_table[0] + p1 * (edge_emb_table[1] - edge_emb_table[0])
    e3 = jnp.concatenate(
        [edge_emb_table, ea_mean[None, :],
         jnp.zeros((5, ED), jnp.float32)], axis=0) @ We.T

    hid = jnp.arange(HD, dtype=jnp.int32)
    h8 = jnp.arange(8, dtype=jnp.int32)
    att_bd = jnp.where((hid[:, None] // D) == h8[None, :],
                       att.reshape(-1)[:, None], 0.0)
    nid = jnp.arange(H * N, dtype=jnp.int32)
    ones_bd32 = ((h8[:, None] < H) & (nid[None, :] // N == h8[:, None])
                 ).astype(jnp.float32) * (1.0 / H)

    return _run(
        x_nodes, src, dst, ew, e3,
        ln1_w.reshape(1, D), ln1_b.reshape(1, D),
        Wl.T, bl.reshape(1, HD), Wr.T, br.reshape(1, HD),
        att_bd, ones_bd32, gat_bias.reshape(1, D),
        ffn_W1.T, ffn_b1.reshape(1, D), ffn_W2.T, ffn_b2.reshape(1, D),
        norm_w.reshape(1, D), norm_b.reshape(1, D))


# speedup vs baseline: 78.3381x; 1.2254x over previous
import functools

import jax
import jax.numpy as jnp
from jax.experimental import pallas as pl
from jax.experimental.pallas import tpu as pltpu

B, Q, N, E, D, H, ED = 16, 64, 32, 64, 128, 4, 8
G = B * Q
HD = H * D

GB = 32


def _bdot(a, b):
    return jax.lax.dot_general(
        a, b,
        dimension_numbers=(((2,), (1,)), ((0,), (0,))),
        preferred_element_type=jnp.float32)


def _bdot16(a, b, out_dtype=jnp.float32):
    return jax.lax.dot_general(
        a.astype(jnp.bfloat16), b.astype(jnp.bfloat16),
        dimension_numbers=(((2,), (1,)), ((0,), (0,))),
        preferred_element_type=out_dtype)


def _dot16(a, b, out_dtype=jnp.float32):
    return jnp.dot(a.astype(jnp.bfloat16), b.astype(jnp.bfloat16),
                   preferred_element_type=out_dtype)


def _gat_kernel(x_ref, src_ref, dst_ref, w_ref, e3_ref,
                ln1w_ref, ln1b_ref, wl_ref, bl_ref, wr_ref, br_ref,
                attbd_ref, onesbd_ref,
                gbias_ref, w1_ref, b1_ref, w2_ref, b2_ref,
                nw_ref, nb_ref, out_ref, xls_ref):
    x = x_ref[...]
    mu = jnp.mean(x, axis=-1, keepdims=True)
    var = jnp.mean((x - mu) ** 2, axis=-1, keepdims=True)
    xn = (x - mu) * jax.lax.rsqrt(var + 1e-5) * ln1w_ref[...] + ln1b_ref[...]

    bf16 = jnp.bfloat16

    xl = (_dot16(xn, wl_ref[...]) + bl_ref[...]).astype(bf16)
    xr = (_dot16(xn, wr_ref[...]) + br_ref[...]).astype(bf16)
    xl_g = xl.reshape(GB, N, HD)
    xr_g = xr.reshape(GB, N, HD)

    src = src_ref[...]
    dst = dst_ref[...]
    iota_n = jax.lax.broadcasted_iota(jnp.int32, (GB, E, N), 2)
    oh_src = (src[:, :, None] == iota_n).astype(bf16)
    oh_dst = (dst[:, :, None] == iota_n).astype(bf16)
    iota_8 = jax.lax.broadcasted_iota(jnp.int32, (GB, E, 8), 2)
    w_oh = (w_ref[...][:, :, None] == iota_8).astype(bf16)

    e3 = e3_ref[...].astype(bf16)
    att_bd = attbd_ref[...].astype(bf16)
    ones_bd32 = onesbd_ref[...].astype(bf16)

    lhs_cat = jnp.concatenate([oh_src, oh_dst, w_oh], axis=-1)
    e3_b = jnp.broadcast_to(e3[None], (GB, 8, HD))
    rhs_cat = jnp.concatenate([xl_g, xr_g, e3_b], axis=1)
    m = _bdot16(lhs_cat, rhs_cat).astype(bf16).reshape(GB * E, HD)

    m = jnp.where(m > 0, m, 0.2 * m)
    logits_e = jnp.dot(m, att_bd,
                       preferred_element_type=jnp.float32).reshape(GB, E, 8)

    ms = (xl + xr) + e3[2][None, :]
    ms = jnp.where(ms > 0, ms, 0.2 * ms)
    logits_s = jnp.dot(ms, att_bd,
                       preferred_element_type=jnp.float32).reshape(GB, N, 8)

    mx_dst = _bdot16(oh_dst, logits_s)
    aexp_e = jnp.exp(logits_e - mx_dst)

    den = _bdot16(oh_dst.transpose(0, 2, 1), aexp_e) + 1.0
    den_dst = _bdot16(oh_dst, den)
    alpha_e = aexp_e / (den_dst + 1e-16)
    alpha_s = 1.0 / (den + 1e-16)

    alpha_lane = _dot16(alpha_e.reshape(GB * E, 8), ones_bd32).astype(bf16)
    oh_src4 = jnp.concatenate([oh_src] * H, axis=-1)
    rhs_a = alpha_lane.reshape(GB, E, H * N) * oh_src4
    a_cat = _bdot16(oh_dst.transpose(0, 2, 1), rhs_a)
    alpha_s_lane = _dot16(alpha_s.reshape(GB * N, 8), ones_bd32)
    r_i = jax.lax.broadcasted_iota(jnp.int32, (N, H * N), 0)
    c_i = jax.lax.broadcasted_iota(jnp.int32, (N, H * N), 1)
    eye4 = (r_i == (c_i % N)).astype(jnp.float32)
    a_cat = (a_cat + eye4[None] * alpha_s_lane.reshape(GB, N, H * N)).astype(bf16)

    for h in range(H):
        xls_ref[:, h * N:(h + 1) * N, :] = xl_g[:, :, h * D:(h + 1) * D]
    out = _bdot16(a_cat, xls_ref[...])
    o = out.reshape(GB * N, D) + gbias_ref[...]

    h1 = _dot16(o, w1_ref[...]) + b1_ref[...]
    h1 = jnp.where(h1 > 0, h1, 0.01 * h1)
    h2 = _dot16(h1, w2_ref[...]) + b2_ref[...]

    gf = jnp.sum(h2.reshape(GB, N, D), axis=1) * (1.0 / N)
    mu2 = jnp.mean(gf, axis=-1, keepdims=True)
    var2 = jnp.mean((gf - mu2) ** 2, axis=-1, keepdims=True)
    out_ref[...] = (gf - mu2) * jax.lax.rsqrt(var2 + 1e-5) * nw_ref[...] + nb_ref[...]


@jax.jit
def _run(x_nodes, src, dst, ew, e3, ln1_w, ln1_b, WlT, bl, WrT, br,
         att_bd, ones_bd32, gat_bias, W1T, b1, W2T, b2, norm_w, norm_b):
    grid = (G // GB,)
    full = lambda shape: pl.BlockSpec(shape, lambda i: (0,) * len(shape))
    out = pl.pallas_call(
        _gat_kernel,
        grid=grid,
        in_specs=[
            pl.BlockSpec((GB * N, D), lambda i: (i, 0)),
            pl.BlockSpec((GB, E), lambda i: (i, 0)),
            pl.BlockSpec((GB, E), lambda i: (i, 0)),
            pl.BlockSpec((GB, E), lambda i: (i, 0)),
            full((8, HD)),
            full((1, D)), full((1, D)),
            full((D, HD)), full((1, HD)),
            full((D, HD)), full((1, HD)),
            full((HD, 8)),
            full((8, H * N)),
            full((1, D)),
            full((D, D)), full((1, D)),
            full((D, D)), full((1, D)),
            full((1, D)), full((1, D)),
        ],
        out_specs=pl.BlockSpec((GB, D), lambda i: (i, 0)),
        out_shape=jax.ShapeDtypeStruct((G, D), jnp.float32),
        scratch_shapes=[pltpu.VMEM((GB, H * N, D), jnp.bfloat16)],
    )(x_nodes, src, dst, ew, e3, ln1_w, ln1_b, WlT, bl, WrT, br,
      att_bd, ones_bd32, gat_bias, W1T, b1, W2T, b2, norm_w, norm_b)
    return out.reshape(B, Q, D)


def kernel(query_embeddings, query_maskings, edge_indexes, edge_weights,
           edge_maskings, edge_emb_table, ln1_w, ln1_b, Wl, bl, Wr, br, We,
           att, gat_bias, ffn_W1, ffn_b1, ffn_W2, ffn_b2, norm_w, norm_b):
    x_nodes = query_embeddings.reshape(G * N, D)
    eidx = edge_indexes.reshape(G, 2, E)
    src = eidx[:, 0, :]
    dst = eidx[:, 1, :]
    ew = edge_weights.reshape(G, E)

    p1 = jnp.mean(ew.astype(jnp.float32))
    ea_mean = edge_emb_table[0] + p1 * (edge_emb_table[1] - edge_emb_table[0])
    e3 = jnp.concatenate(
        [edge_emb_table, ea_mean[None, :],
         jnp.zeros((5, ED), jnp.float32)], axis=0) @ We.T

    hid = jnp.arange(HD, dtype=jnp.int32)
    h8 = jnp.arange(8, dtype=jnp.int32)
    att_bd = jnp.where((hid[:, None] // D) == h8[None, :],
                       att.reshape(-1)[:, None], 0.0)
    nid = jnp.arange(H * N, dtype=jnp.int32)
    ones_bd32 = ((h8[:, None] < H) & (nid[None, :] // N == h8[:, None])
                 ).astype(jnp.float32) * (1.0 / H)

    return _run(
        x_nodes, src, dst, ew, e3,
        ln1_w.reshape(1, D), ln1_b.reshape(1, D),
        Wl.T, bl.reshape(1, HD), Wr.T, br.reshape(1, HD),
        att_bd, ones_bd32, gat_bias.reshape(1, D),
        ffn_W1.T, ffn_b1.reshape(1, D), ffn_W2.T, ffn_b2.reshape(1, D),
        norm_w.reshape(1, D), norm_b.reshape(1, D))


# speedup vs baseline: 85.4704x; 1.0910x over previous
import functools

import jax
import jax.numpy as jnp
from jax.experimental import pallas as pl
from jax.experimental.pallas import tpu as pltpu

B, Q, N, E, D, H, ED = 16, 64, 32, 64, 128, 4, 8
G = B * Q
HD = H * D

GB = 64


def _bdot(a, b):
    return jax.lax.dot_general(
        a, b,
        dimension_numbers=(((2,), (1,)), ((0,), (0,))),
        preferred_element_type=jnp.float32)


def _bdot16(a, b, out_dtype=jnp.float32):
    return jax.lax.dot_general(
        a.astype(jnp.bfloat16), b.astype(jnp.bfloat16),
        dimension_numbers=(((2,), (1,)), ((0,), (0,))),
        preferred_element_type=out_dtype)


def _dot16(a, b, out_dtype=jnp.float32):
    return jnp.dot(a.astype(jnp.bfloat16), b.astype(jnp.bfloat16),
                   preferred_element_type=out_dtype)


def _gat_kernel(x_ref, src_ref, dst_ref, w_ref, e3_ref,
                ln1w_ref, ln1b_ref, wl_ref, bl_ref, wr_ref, br_ref,
                attbd_ref, onesbd_ref,
                gbias_ref, w1_ref, b1_ref, w2_ref, b2_ref,
                nw_ref, nb_ref, out_ref, xls_ref):
    x = x_ref[...]
    mu = jnp.mean(x, axis=-1, keepdims=True)
    var = jnp.mean((x - mu) ** 2, axis=-1, keepdims=True)
    xn = (x - mu) * jax.lax.rsqrt(var + 1e-5) * ln1w_ref[...] + ln1b_ref[...]

    bf16 = jnp.bfloat16

    xl = (_dot16(xn, wl_ref[...]) + bl_ref[...]).astype(bf16)
    xr = (_dot16(xn, wr_ref[...]) + br_ref[...]).astype(bf16)
    xl_g = xl.reshape(GB, N, HD)
    xr_g = xr.reshape(GB, N, HD)

    src = src_ref[...]
    dst = dst_ref[...]
    iota_n = jax.lax.broadcasted_iota(jnp.int32, (GB, E, N), 2)
    oh_src = (src[:, :, None] == iota_n).astype(bf16)
    oh_dst = (dst[:, :, None] == iota_n).astype(bf16)
    iota_8 = jax.lax.broadcasted_iota(jnp.int32, (GB, E, 8), 2)
    w_oh = (w_ref[...][:, :, None] == iota_8).astype(bf16)

    e3 = e3_ref[...].astype(bf16)
    att_bd = attbd_ref[...].astype(bf16)
    ones_bd32 = onesbd_ref[...].astype(bf16)

    lhs_cat = jnp.concatenate([oh_src, oh_dst, w_oh], axis=-1)
    e3_b = jnp.broadcast_to(e3[None], (GB, 8, HD))
    rhs_cat = jnp.concatenate([xl_g, xr_g, e3_b], axis=1)
    m = _bdot16(lhs_cat, rhs_cat).astype(bf16).reshape(GB * E, HD)

    m = jnp.where(m > 0, m, 0.2 * m)
    logits_e = jnp.dot(m, att_bd,
                       preferred_element_type=jnp.float32).reshape(GB, E, 8)

    ms = (xl + xr) + e3[2][None, :]
    ms = jnp.where(ms > 0, ms, 0.2 * ms)
    logits_s = jnp.dot(ms, att_bd,
                       preferred_element_type=jnp.float32).reshape(GB, N, 8)

    mx_dst = _bdot16(oh_dst, logits_s)
    aexp_e = jnp.exp(logits_e - mx_dst)

    den = _bdot16(oh_dst.transpose(0, 2, 1), aexp_e) + 1.0
    den_dst = _bdot16(oh_dst, den)
    alpha_e = aexp_e / (den_dst + 1e-16)
    alpha_s = 1.0 / (den + 1e-16)

    alpha_lane = _dot16(alpha_e.reshape(GB * E, 8), ones_bd32).astype(bf16)
    oh_src4 = jnp.concatenate([oh_src] * H, axis=-1)
    rhs_a = alpha_lane.reshape(GB, E, H * N) * oh_src4
    a_cat = _bdot16(oh_dst.transpose(0, 2, 1), rhs_a)
    alpha_s_lane = _dot16(alpha_s.reshape(GB * N, 8), ones_bd32)
    r_i = jax.lax.broadcasted_iota(jnp.int32, (N, H * N), 0)
    c_i = jax.lax.broadcasted_iota(jnp.int32, (N, H * N), 1)
    eye4 = (r_i == (c_i % N)).astype(jnp.float32)
    a_cat = (a_cat + eye4[None] * alpha_s_lane.reshape(GB, N, H * N)).astype(bf16)

    for h in range(H):
        xls_ref[:, h * N:(h + 1) * N, :] = xl_g[:, :, h * D:(h + 1) * D]
    out = _bdot16(a_cat, xls_ref[...])
    o = out.reshape(GB * N, D) + gbias_ref[...]

    h1 = _dot16(o, w1_ref[...]) + b1_ref[...]
    h1 = jnp.where(h1 > 0, h1, 0.01 * h1)
    h2 = _dot16(h1, w2_ref[...]) + b2_ref[...]

    gf = jnp.sum(h2.reshape(GB, N, D), axis=1) * (1.0 / N)
    mu2 = jnp.mean(gf, axis=-1, keepdims=True)
    var2 = jnp.mean((gf - mu2) ** 2, axis=-1, keepdims=True)
    out_ref[...] = (gf - mu2) * jax.lax.rsqrt(var2 + 1e-5) * nw_ref[...] + nb_ref[...]


@jax.jit
def _run(x_nodes, src, dst, ew, e3, ln1_w, ln1_b, WlT, bl, WrT, br,
         att_bd, ones_bd32, gat_bias, W1T, b1, W2T, b2, norm_w, norm_b):
    grid = (G // GB,)
    full = lambda shape: pl.BlockSpec(shape, lambda i: (0,) * len(shape))
    out = pl.pallas_call(
        _gat_kernel,
        grid=grid,
        in_specs=[
            pl.BlockSpec((GB * N, D), lambda i: (i, 0)),
            pl.BlockSpec((GB, E), lambda i: (i, 0)),
            pl.BlockSpec((GB, E), lambda i: (i, 0)),
            pl.BlockSpec((GB, E), lambda i: (i, 0)),
            full((8, HD)),
            full((1, D)), full((1, D)),
            full((D, HD)), full((1, HD)),
            full((D, HD)), full((1, HD)),
            full((HD, 8)),
            full((8, H * N)),
            full((1, D)),
            full((D, D)), full((1, D)),
            full((D, D)), full((1, D)),
            full((1, D)), full((1, D)),
        ],
        out_specs=pl.BlockSpec((GB, D), lambda i: (i, 0)),
        out_shape=jax.ShapeDtypeStruct((G, D), jnp.float32),
        scratch_shapes=[pltpu.VMEM((GB, H * N, D), jnp.bfloat16)],
    )(x_nodes, src, dst, ew, e3, ln1_w, ln1_b, WlT, bl, WrT, br,
      att_bd, ones_bd32, gat_bias, W1T, b1, W2T, b2, norm_w, norm_b)
    return out.reshape(B, Q, D)


def kernel(query_embeddings, query_maskings, edge_indexes, edge_weights,
           edge_maskings, edge_emb_table, ln1_w, ln1_b, Wl, bl, Wr, br, We,
           att, gat_bias, ffn_W1, ffn_b1, ffn_W2, ffn_b2, norm_w, norm_b):
    x_nodes = query_embeddings.reshape(G * N, D)
    eidx = edge_indexes.reshape(G, 2, E)
    src = eidx[:, 0, :]
    dst = eidx[:, 1, :]
    ew = edge_weights.reshape(G, E)

    p1 = jnp.mean(ew.astype(jnp.float32))
    ea_mean = edge_emb_table[0] + p1 * (edge_emb_table[1] - edge_emb_table[0])
    e3 = jnp.concatenate(
        [edge_emb_table, ea_mean[None, :],
         jnp.zeros((5, ED), jnp.float32)], axis=0) @ We.T

    hid = jnp.arange(HD, dtype=jnp.int32)
    h8 = jnp.arange(8, dtype=jnp.int32)
    att_bd = jnp.where((hid[:, None] // D) == h8[None, :],
                       att.reshape(-1)[:, None], 0.0)
    nid = jnp.arange(H * N, dtype=jnp.int32)
    ones_bd32 = ((h8[:, None] < H) & (nid[None, :] // N == h8[:, None])
                 ).astype(jnp.float32) * (1.0 / H)

    return _run(
        x_nodes, src, dst, ew, e3,
        ln1_w.reshape(1, D), ln1_b.reshape(1, D),
        Wl.T, bl.reshape(1, HD), Wr.T, br.reshape(1, HD),
        att_bd, ones_bd32, gat_bias.reshape(1, D),
        ffn_W1.T, ffn_b1.reshape(1, D), ffn_W2.T, ffn_b2.reshape(1, D),
        norm_w.reshape(1, D), norm_b.reshape(1, D))


# speedup vs baseline: 89.0447x; 1.0418x over previous
import functools

import jax
import jax.numpy as jnp
from jax.experimental import pallas as pl
from jax.experimental.pallas import tpu as pltpu

B, Q, N, E, D, H, ED = 16, 64, 32, 64, 128, 4, 8
G = B * Q
HD = H * D

GB = 128


def _bdot(a, b):
    return jax.lax.dot_general(
        a, b,
        dimension_numbers=(((2,), (1,)), ((0,), (0,))),
        preferred_element_type=jnp.float32)


def _bdot16(a, b, out_dtype=jnp.float32):
    return jax.lax.dot_general(
        a.astype(jnp.bfloat16), b.astype(jnp.bfloat16),
        dimension_numbers=(((2,), (1,)), ((0,), (0,))),
        preferred_element_type=out_dtype)


def _dot16(a, b, out_dtype=jnp.float32):
    return jnp.dot(a.astype(jnp.bfloat16), b.astype(jnp.bfloat16),
                   preferred_element_type=out_dtype)


def _gat_kernel(x_ref, src_ref, dst_ref, w_ref, e3_ref,
                ln1w_ref, ln1b_ref, wl_ref, bl_ref, wr_ref, br_ref,
                attbd_ref, onesbd_ref,
                gbias_ref, w1_ref, b1_ref, w2_ref, b2_ref,
                nw_ref, nb_ref, out_ref, xls_ref):
    x = x_ref[...]
    mu = jnp.mean(x, axis=-1, keepdims=True)
    var = jnp.mean((x - mu) ** 2, axis=-1, keepdims=True)
    xn = (x - mu) * jax.lax.rsqrt(var + 1e-5) * ln1w_ref[...] + ln1b_ref[...]

    bf16 = jnp.bfloat16

    xl = (_dot16(xn, wl_ref[...]) + bl_ref[...]).astype(bf16)
    xr = (_dot16(xn, wr_ref[...]) + br_ref[...]).astype(bf16)
    xl_g = xl.reshape(GB, N, HD)
    xr_g = xr.reshape(GB, N, HD)

    src = src_ref[...]
    dst = dst_ref[...]
    iota_n = jax.lax.broadcasted_iota(jnp.int32, (GB, E, N), 2)
    oh_src = (src[:, :, None] == iota_n).astype(bf16)
    oh_dst = (dst[:, :, None] == iota_n).astype(bf16)
    iota_8 = jax.lax.broadcasted_iota(jnp.int32, (GB, E, 8), 2)
    w_oh = (w_ref[...][:, :, None] == iota_8).astype(bf16)

    e3 = e3_ref[...].astype(bf16)
    att_bd = attbd_ref[...].astype(bf16)
    ones_bd32 = onesbd_ref[...].astype(bf16)

    lhs_cat = jnp.concatenate([oh_src, oh_dst, w_oh], axis=-1)
    e3_b = jnp.broadcast_to(e3[None], (GB, 8, HD))
    rhs_cat = jnp.concatenate([xl_g, xr_g, e3_b], axis=1)
    m = _bdot16(lhs_cat, rhs_cat).astype(bf16).reshape(GB * E, HD)

    m = jnp.where(m > 0, m, 0.2 * m)
    logits_e = jnp.dot(m, att_bd,
                       preferred_element_type=jnp.float32).reshape(GB, E, 8)

    ms = (xl + xr) + e3[2][None, :]
    ms = jnp.where(ms > 0, ms, 0.2 * ms)
    logits_s = jnp.dot(ms, att_bd,
                       preferred_element_type=jnp.float32).reshape(GB, N, 8)

    mx_dst = _bdot16(oh_dst, logits_s)
    aexp_e = jnp.exp(logits_e - mx_dst)

    den = _bdot16(oh_dst.transpose(0, 2, 1), aexp_e) + 1.0
    den_dst = _bdot16(oh_dst, den)
    alpha_e = aexp_e / (den_dst + 1e-16)
    alpha_s = 1.0 / (den + 1e-16)

    alpha_lane = _dot16(alpha_e.reshape(GB * E, 8), ones_bd32).astype(bf16)
    oh_src4 = jnp.concatenate([oh_src] * H, axis=-1)
    rhs_a = alpha_lane.reshape(GB, E, H * N) * oh_src4
    a_cat = _bdot16(oh_dst.transpose(0, 2, 1), rhs_a)
    alpha_s_lane = _dot16(alpha_s.reshape(GB * N, 8), ones_bd32)
    r_i = jax.lax.broadcasted_iota(jnp.int32, (N, H * N), 0)
    c_i = jax.lax.broadcasted_iota(jnp.int32, (N, H * N), 1)
    eye4 = (r_i == (c_i % N)).astype(jnp.float32)
    a_cat = (a_cat + eye4[None] * alpha_s_lane.reshape(GB, N, H * N)).astype(bf16)

    for h in range(H):
        xls_ref[:, h * N:(h + 1) * N, :] = xl_g[:, :, h * D:(h + 1) * D]
    out = _bdot16(a_cat, xls_ref[...])
    o = out.reshape(GB * N, D) + gbias_ref[...]

    h1 = _dot16(o, w1_ref[...]) + b1_ref[...]
    h1 = jnp.where(h1 > 0, h1, 0.01 * h1)
    h2 = _dot16(h1, w2_ref[...]) + b2_ref[...]

    gf = jnp.sum(h2.reshape(GB, N, D), axis=1) * (1.0 / N)
    mu2 = jnp.mean(gf, axis=-1, keepdims=True)
    var2 = jnp.mean((gf - mu2) ** 2, axis=-1, keepdims=True)
    out_ref[...] = (gf - mu2) * jax.lax.rsqrt(var2 + 1e-5) * nw_ref[...] + nb_ref[...]


@jax.jit
def _run(x_nodes, src, dst, ew, e3, ln1_w, ln1_b, WlT, bl, WrT, br,
         att_bd, ones_bd32, gat_bias, W1T, b1, W2T, b2, norm_w, norm_b):
    grid = (G // GB,)
    full = lambda shape: pl.BlockSpec(shape, lambda i: (0,) * len(shape))
    out = pl.pallas_call(
        _gat_kernel,
        grid=grid,
        in_specs=[
            pl.BlockSpec((GB * N, D), lambda i: (i, 0)),
            pl.BlockSpec((GB, E), lambda i: (i, 0)),
            pl.BlockSpec((GB, E), lambda i: (i, 0)),
            pl.BlockSpec((GB, E), lambda i: (i, 0)),
            full((8, HD)),
            full((1, D)), full((1, D)),
            full((D, HD)), full((1, HD)),
            full((D, HD)), full((1, HD)),
            full((HD, 8)),
            full((8, H * N)),
            full((1, D)),
            full((D, D)), full((1, D)),
            full((D, D)), full((1, D)),
            full((1, D)), full((1, D)),
        ],
        out_specs=pl.BlockSpec((GB, D), lambda i: (i, 0)),
        out_shape=jax.ShapeDtypeStruct((G, D), jnp.float32),
        scratch_shapes=[pltpu.VMEM((GB, H * N, D), jnp.bfloat16)],
    )(x_nodes, src, dst, ew, e3, ln1_w, ln1_b, WlT, bl, WrT, br,
      att_bd, ones_bd32, gat_bias, W1T, b1, W2T, b2, norm_w, norm_b)
    return out.reshape(B, Q, D)


def kernel(query_embeddings, query_maskings, edge_indexes, edge_weights,
           edge_maskings, edge_emb_table, ln1_w, ln1_b, Wl, bl, Wr, br, We,
           att, gat_bias, ffn_W1, ffn_b1, ffn_W2, ffn_b2, norm_w, norm_b):
    x_nodes = query_embeddings.reshape(G * N, D)
    eidx = edge_indexes.reshape(G, 2, E)
    src = eidx[:, 0, :]
    dst = eidx[:, 1, :]
    ew = edge_weights.reshape(G, E)

    p1 = jnp.mean(ew.astype(jnp.float32))
    ea_mean = edge_emb_table[0] + p1 * (edge_emb_table[1] - edge_emb_table[0])
    e3 = jnp.concatenate(
        [edge_emb_table, ea_mean[None, :],
         jnp.zeros((5, ED), jnp.float32)], axis=0) @ We.T

    hid = jnp.arange(HD, dtype=jnp.int32)
    h8 = jnp.arange(8, dtype=jnp.int32)
    att_bd = jnp.where((hid[:, None] // D) == h8[None, :],
                       att.reshape(-1)[:, None], 0.0)
    nid = jnp.arange(H * N, dtype=jnp.int32)
    ones_bd32 = ((h8[:, None] < H) & (nid[None, :] // N == h8[:, None])
                 ).astype(jnp.float32) * (1.0 / H)

    return _run(
        x_nodes, src, dst, ew, e3,
        ln1_w.reshape(1, D), ln1_b.reshape(1, D),
        Wl.T, bl.reshape(1, HD), Wr.T, br.reshape(1, HD),
        att_bd, ones_bd32, gat_bias.reshape(1, D),
        ffn_W1.T, ffn_b1.reshape(1, D), ffn_W2.T, ffn_b2.reshape(1, D),
        norm_w.reshape(1, D), norm_b.reshape(1, D))


# speedup vs baseline: 94.7848x; 1.0645x over previous
import functools

import jax
import jax.numpy as jnp
from jax.experimental import pallas as pl
from jax.experimental.pallas import tpu as pltpu

B, Q, N, E, D, H, ED = 16, 64, 32, 64, 128, 4, 8
G = B * Q
HD = H * D

GB = 128


def _bdot(a, b):
    return jax.lax.dot_general(
        a, b,
        dimension_numbers=(((2,), (1,)), ((0,), (0,))),
        preferred_element_type=jnp.float32)


def _bdot16(a, b, out_dtype=jnp.float32):
    return jax.lax.dot_general(
        a.astype(jnp.bfloat16), b.astype(jnp.bfloat16),
        dimension_numbers=(((2,), (1,)), ((0,), (0,))),
        preferred_element_type=out_dtype)


def _dot16(a, b, out_dtype=jnp.float32):
    return jnp.dot(a.astype(jnp.bfloat16), b.astype(jnp.bfloat16),
                   preferred_element_type=out_dtype)


def _gat_kernel(x_ref, src_ref, dst_ref, w_ref, e3_ref,
                ln1w_ref, ln1b_ref, wl_ref, bl_ref, wr_ref, br_ref,
                attbd_ref, onesbd_ref,
                gbias_ref, w1_ref, b1_ref, w2_ref, b2_ref,
                nw_ref, nb_ref, out_ref, xls_ref):
    x = x_ref[...]
    mu = jnp.mean(x, axis=-1, keepdims=True)
    var = jnp.mean((x - mu) ** 2, axis=-1, keepdims=True)
    xn = (x - mu) * jax.lax.rsqrt(var + 1e-5) * ln1w_ref[...] + ln1b_ref[...]

    bf16 = jnp.bfloat16

    xl = (_dot16(xn, wl_ref[...]) + bl_ref[...]).astype(bf16)
    xr = (_dot16(xn, wr_ref[...]) + br_ref[...]).astype(bf16)
    xl_g = xl.reshape(GB, N, HD)
    xr_g = xr.reshape(GB, N, HD)

    src = src_ref[...]
    dst = dst_ref[...]
    srcb = src[:, :, None]
    dstb = dst[:, :, None]
    wb = w_ref[...][:, :, None]
    iota_n = jax.lax.broadcasted_iota(jnp.int32, (GB, E, N), 2)
    oh_dst = (dstb == iota_n).astype(bf16)
    iota_sl = jax.lax.broadcasted_iota(jnp.int32, (GB, N, E), 1)
    oh_dst_t = (dst[:, None, :] == iota_sl).astype(bf16)
    l72 = jax.lax.broadcasted_iota(jnp.int32, (GB, E, 2 * N + 8), 2)
    tgt = jnp.where(l72 < N, srcb,
                    jnp.where(l72 < 2 * N, dstb + N, wb + 2 * N))
    lhs_cat = (l72 == tgt).astype(bf16)
    l128 = jax.lax.broadcasted_iota(jnp.int32, (GB, E, H * N), 2)
    oh_src4 = ((l128 & (N - 1)) == srcb).astype(bf16)

    e3 = e3_ref[...].astype(bf16)
    att_bd = attbd_ref[...].astype(bf16)
    ones_bd32 = onesbd_ref[...].astype(bf16)

    e3_b = jnp.broadcast_to(e3[None], (GB, 8, HD))
    rhs_cat = jnp.concatenate([xl_g, xr_g, e3_b], axis=1)
    m = _bdot16(lhs_cat, rhs_cat).astype(bf16).reshape(GB * E, HD)

    m = jnp.where(m > 0, m, 0.2 * m)
    logits_e = jnp.dot(m, att_bd,
                       preferred_element_type=jnp.float32).reshape(GB, E, 8)

    ms = (xl + xr) + e3[2][None, :]
    ms = jnp.where(ms > 0, ms, 0.2 * ms)
    logits_s = jnp.dot(ms, att_bd,
                       preferred_element_type=jnp.float32).reshape(GB, N, 8)

    mx_dst = _bdot16(oh_dst, logits_s)
    aexp_e = jnp.exp(logits_e - mx_dst)

    den = _bdot16(oh_dst_t, aexp_e) + 1.0
    den_dst = _bdot16(oh_dst, den)
    alpha_e = aexp_e / (den_dst + 1e-16)
    alpha_s = 1.0 / (den + 1e-16)

    alpha_lane = _dot16(alpha_e.reshape(GB * E, 8), ones_bd32).astype(bf16)
    rhs_a = alpha_lane.reshape(GB, E, H * N) * oh_src4
    a_cat = _bdot16(oh_dst_t, rhs_a)
    alpha_s_lane = _dot16(alpha_s.reshape(GB * N, 8), ones_bd32)
    r_i = jax.lax.broadcasted_iota(jnp.int32, (N, H * N), 0)
    c_i = jax.lax.broadcasted_iota(jnp.int32, (N, H * N), 1)
    eye4 = (r_i == (c_i % N)).astype(jnp.float32)
    a_cat = (a_cat + eye4[None] * alpha_s_lane.reshape(GB, N, H * N)).astype(bf16)

    for h in range(H):
        xls_ref[:, h * N:(h + 1) * N, :] = xl_g[:, :, h * D:(h + 1) * D]
    out = _bdot16(a_cat, xls_ref[...])
    o = out.reshape(GB * N, D) + gbias_ref[...]

    h1 = _dot16(o, w1_ref[...]) + b1_ref[...]
    h1 = jnp.where(h1 > 0, h1, 0.01 * h1)
    h2 = _dot16(h1, w2_ref[...]) + b2_ref[...]

    gf = jnp.sum(h2.reshape(GB, N, D), axis=1) * (1.0 / N)
    mu2 = jnp.mean(gf, axis=-1, keepdims=True)
    var2 = jnp.mean((gf - mu2) ** 2, axis=-1, keepdims=True)
    out_ref[...] = (gf - mu2) * jax.lax.rsqrt(var2 + 1e-5) * nw_ref[...] + nb_ref[...]


@jax.jit
def _run(x_nodes, src, dst, ew, e3, ln1_w, ln1_b, WlT, bl, WrT, br,
         att_bd, ones_bd32, gat_bias, W1T, b1, W2T, b2, norm_w, norm_b):
    grid = (G // GB,)
    full = lambda shape: pl.BlockSpec(shape, lambda i: (0,) * len(shape))
    out = pl.pallas_call(
        _gat_kernel,
        grid=grid,
        in_specs=[
            pl.BlockSpec((GB * N, D), lambda i: (i, 0)),
            pl.BlockSpec((GB, E), lambda i: (i, 0)),
            pl.BlockSpec((GB, E), lambda i: (i, 0)),
            pl.BlockSpec((GB, E), lambda i: (i, 0)),
            full((8, HD)),
            full((1, D)), full((1, D)),
            full((D, HD)), full((1, HD)),
            full((D, HD)), full((1, HD)),
            full((HD, 8)),
            full((8, H * N)),
            full((1, D)),
            full((D, D)), full((1, D)),
            full((D, D)), full((1, D)),
            full((1, D)), full((1, D)),
        ],
        out_specs=pl.BlockSpec((GB, D), lambda i: (i, 0)),
        out_shape=jax.ShapeDtypeStruct((G, D), jnp.float32),
        scratch_shapes=[pltpu.VMEM((GB, H * N, D), jnp.bfloat16)],
    )(x_nodes, src, dst, ew, e3, ln1_w, ln1_b, WlT, bl, WrT, br,
      att_bd, ones_bd32, gat_bias, W1T, b1, W2T, b2, norm_w, norm_b)
    return out.reshape(B, Q, D)


def kernel(query_embeddings, query_maskings, edge_indexes, edge_weights,
           edge_maskings, edge_emb_table, ln1_w, ln1_b, Wl, bl, Wr, br, We,
           att, gat_bias, ffn_W1, ffn_b1, ffn_W2, ffn_b2, norm_w, norm_b):
    x_nodes = query_embeddings.reshape(G * N, D)
    eidx = edge_indexes.reshape(G, 2, E)
    src = eidx[:, 0, :]
    dst = eidx[:, 1, :]
    ew = edge_weights.reshape(G, E)

    p1 = jnp.mean(ew.astype(jnp.float32))
    ea_mean = edge_emb_table[0] + p1 * (edge_emb_table[1] - edge_emb_table[0])
    e3 = jnp.concatenate(
        [edge_emb_table, ea_mean[None, :],
         jnp.zeros((5, ED), jnp.float32)], axis=0) @ We.T

    hid = jnp.arange(HD, dtype=jnp.int32)
    h8 = jnp.arange(8, dtype=jnp.int32)
    att_bd = jnp.where((hid[:, None] // D) == h8[None, :],
                       att.reshape(-1)[:, None], 0.0)
    nid = jnp.arange(H * N, dtype=jnp.int32)
    ones_bd32 = ((h8[:, None] < H) & (nid[None, :] // N == h8[:, None])
                 ).astype(jnp.float32) * (1.0 / H)

    return _run(
        x_nodes, src, dst, ew, e3,
        ln1_w.reshape(1, D), ln1_b.reshape(1, D),
        Wl.T, bl.reshape(1, HD), Wr.T, br.reshape(1, HD),
        att_bd, ones_bd32, gat_bias.reshape(1, D),
        ffn_W1.T, ffn_b1.reshape(1, D), ffn_W2.T, ffn_b2.reshape(1, D),
        norm_w.reshape(1, D), norm_b.reshape(1, D))


# speedup vs baseline: 95.8537x; 1.0113x over previous
import functools

import jax
import jax.numpy as jnp
from jax.experimental import pallas as pl
from jax.experimental.pallas import tpu as pltpu

B, Q, N, E, D, H, ED = 16, 64, 32, 64, 128, 4, 8
G = B * Q
HD = H * D

GB = 128


def _bdot(a, b):
    return jax.lax.dot_general(
        a, b,
        dimension_numbers=(((2,), (1,)), ((0,), (0,))),
        preferred_element_type=jnp.float32)


def _bdot16(a, b, out_dtype=jnp.float32):
    return jax.lax.dot_general(
        a.astype(jnp.bfloat16), b.astype(jnp.bfloat16),
        dimension_numbers=(((2,), (1,)), ((0,), (0,))),
        preferred_element_type=out_dtype)


def _dot16(a, b, out_dtype=jnp.float32):
    return jnp.dot(a.astype(jnp.bfloat16), b.astype(jnp.bfloat16),
                   preferred_element_type=out_dtype)


def _gat_kernel(x_ref, src_ref, dst_ref, w_ref, e3_ref,
                ln1w_ref, ln1b_ref, wl_ref, bl_ref, wr_ref, br_ref,
                attbd_ref, onesbd_ref,
                gbias_ref, w1_ref, b1_ref, w2_ref, b2_ref,
                nw_ref, nb_ref, out_ref, xls_ref):
    bf16 = jnp.bfloat16

    x = x_ref[...]
    mu = jnp.mean(x, axis=-1, keepdims=True)
    d = (x - mu).astype(bf16)
    var = jnp.mean(d * d, axis=-1, keepdims=True)
    rs = jax.lax.rsqrt(var.astype(jnp.float32) + 1e-5).astype(bf16)
    xn = (d * rs) * ln1w_ref[...].astype(bf16) + ln1b_ref[...].astype(bf16)

    xl = (_dot16(xn, wl_ref[...]) + bl_ref[...]).astype(bf16)
    xr = (_dot16(xn, wr_ref[...]) + br_ref[...]).astype(bf16)
    xl_g = xl.reshape(GB, N, HD)
    xr_g = xr.reshape(GB, N, HD)

    src = src_ref[...]
    dst = dst_ref[...]
    srcb = src[:, :, None]
    dstb = dst[:, :, None]
    wb = w_ref[...][:, :, None]
    iota_n = jax.lax.broadcasted_iota(jnp.int32, (GB, E, N), 2)
    oh_dst = (dstb == iota_n).astype(bf16)
    iota_sl = jax.lax.broadcasted_iota(jnp.int32, (GB, N, E), 1)
    oh_dst_t = (dst[:, None, :] == iota_sl).astype(bf16)
    l72 = jax.lax.broadcasted_iota(jnp.int32, (GB, E, 2 * N + 8), 2)
    tgt = jnp.where(l72 < N, srcb,
                    jnp.where(l72 < 2 * N, dstb + N, wb + 2 * N))
    lhs_cat = (l72 == tgt).astype(bf16)
    l128 = jax.lax.broadcasted_iota(jnp.int32, (GB, E, H * N), 2)
    oh_src4 = ((l128 & (N - 1)) == srcb).astype(bf16)

    e3 = e3_ref[...].astype(bf16)
    att_bd = attbd_ref[...].astype(bf16)
    ones_bd32 = onesbd_ref[...].astype(bf16)

    e3_b = jnp.broadcast_to(e3[None], (GB, 8, HD))
    rhs_cat = jnp.concatenate([xl_g, xr_g, e3_b], axis=1)
    m = _bdot16(lhs_cat, rhs_cat).astype(bf16).reshape(GB * E, HD)

    m = jnp.where(m > 0, m, 0.2 * m)
    logits_e = jnp.dot(m, att_bd,
                       preferred_element_type=jnp.float32).reshape(GB, E, 8)

    ms = (xl + xr) + e3[2][None, :]
    ms = jnp.where(ms > 0, ms, 0.2 * ms)
    logits_s = jnp.dot(ms, att_bd,
                       preferred_element_type=jnp.float32).reshape(GB, N, 8)

    mx_dst = _bdot16(oh_dst, logits_s)
    aexp_e = jnp.exp(logits_e - mx_dst)

    den = _bdot16(oh_dst_t, aexp_e) + 1.0
    den_dst = _bdot16(oh_dst, den)
    alpha_e = aexp_e / (den_dst + 1e-16)
    alpha_s = 1.0 / (den + 1e-16)

    alpha_lane = _dot16(alpha_e.reshape(GB * E, 8), ones_bd32).astype(bf16)
    rhs_a = alpha_lane.reshape(GB, E, H * N) * oh_src4
    a_cat = _bdot16(oh_dst_t, rhs_a)
    alpha_s_lane = _dot16(alpha_s.reshape(GB * N, 8), ones_bd32)
    r_i = jax.lax.broadcasted_iota(jnp.int32, (N, H * N), 0)
    c_i = jax.lax.broadcasted_iota(jnp.int32, (N, H * N), 1)
    eye4 = (r_i == (c_i % N)).astype(jnp.float32)
    a_cat = (a_cat + eye4[None] * alpha_s_lane.reshape(GB, N, H * N)).astype(bf16)

    for h in range(H):
        xls_ref[:, h * N:(h + 1) * N, :] = xl_g[:, :, h * D:(h + 1) * D]
    out = _bdot16(a_cat, xls_ref[...])
    o = out.reshape(GB * N, D) + gbias_ref[...]

    h1 = _dot16(o, w1_ref[...]) + b1_ref[...]
    h1 = jnp.where(h1 > 0, h1, 0.01 * h1)
    h2 = _dot16(h1, w2_ref[...]) + b2_ref[...]

    gf = jnp.sum(h2.reshape(GB, N, D), axis=1) * (1.0 / N)
    mu2 = jnp.mean(gf, axis=-1, keepdims=True)
    var2 = jnp.mean((gf - mu2) ** 2, axis=-1, keepdims=True)
    out_ref[...] = (gf - mu2) * jax.lax.rsqrt(var2 + 1e-5) * nw_ref[...] + nb_ref[...]


@jax.jit
def _run(x_nodes, src, dst, ew, e3, ln1_w, ln1_b, WlT, bl, WrT, br,
         att_bd, ones_bd32, gat_bias, W1T, b1, W2T, b2, norm_w, norm_b):
    grid = (G // GB,)
    full = lambda shape: pl.BlockSpec(shape, lambda i: (0,) * len(shape))
    out = pl.pallas_call(
        _gat_kernel,
        grid=grid,
        in_specs=[
            pl.BlockSpec((GB * N, D), lambda i: (i, 0)),
            pl.BlockSpec((GB, E), lambda i: (i, 0)),
            pl.BlockSpec((GB, E), lambda i: (i, 0)),
            pl.BlockSpec((GB, E), lambda i: (i, 0)),
            full((8, HD)),
            full((1, D)), full((1, D)),
            full((D, HD)), full((1, HD)),
            full((D, HD)), full((1, HD)),
            full((HD, 8)),
            full((8, H * N)),
            full((1, D)),
            full((D, D)), full((1, D)),
            full((D, D)), full((1, D)),
            full((1, D)), full((1, D)),
        ],
        out_specs=pl.BlockSpec((GB, D), lambda i: (i, 0)),
        out_shape=jax.ShapeDtypeStruct((G, D), jnp.float32),
        scratch_shapes=[pltpu.VMEM((GB, H * N, D), jnp.bfloat16)],
    )(x_nodes, src, dst, ew, e3, ln1_w, ln1_b, WlT, bl, WrT, br,
      att_bd, ones_bd32, gat_bias, W1T, b1, W2T, b2, norm_w, norm_b)
    return out.reshape(B, Q, D)


def kernel(query_embeddings, query_maskings, edge_indexes, edge_weights,
           edge_maskings, edge_emb_table, ln1_w, ln1_b, Wl, bl, Wr, br, We,
           att, gat_bias, ffn_W1, ffn_b1, ffn_W2, ffn_b2, norm_w, norm_b):
    x_nodes = query_embeddings.reshape(G * N, D)
    eidx = edge_indexes.reshape(G, 2, E)
    src = eidx[:, 0, :]
    dst = eidx[:, 1, :]
    ew = edge_weights.reshape(G, E)

    p1 = jnp.mean(ew.astype(jnp.float32))
    ea_mean = edge_emb_table[0] + p1 * (edge_emb_table[1] - edge_emb_table[0])
    e3 = jnp.concatenate(
        [edge_emb_table, ea_mean[None, :],
         jnp.zeros((5, ED), jnp.float32)], axis=0) @ We.T

    hid = jnp.arange(HD, dtype=jnp.int32)
    h8 = jnp.arange(8, dtype=jnp.int32)
    att_bd = jnp.where((hid[:, None] // D) == h8[None, :],
                       att.reshape(-1)[:, None], 0.0)
    nid = jnp.arange(H * N, dtype=jnp.int32)
    ones_bd32 = ((h8[:, None] < H) & (nid[None, :] // N == h8[:, None])
                 ).astype(jnp.float32) * (1.0 / H)

    return _run(
        x_nodes, src, dst, ew, e3,
        ln1_w.reshape(1, D), ln1_b.reshape(1, D),
        Wl.T, bl.reshape(1, HD), Wr.T, br.reshape(1, HD),
        att_bd, ones_bd32, gat_bias.reshape(1, D),
        ffn_W1.T, ffn_b1.reshape(1, D), ffn_W2.T, ffn_b2.reshape(1, D),
        norm_w.reshape(1, D), norm_b.reshape(1, D))


# speedup vs baseline: 95.9465x; 1.0010x over previous
import jax
import jax.numpy as jnp
from jax.experimental import pallas as pl
from jax.experimental.pallas import tpu as pltpu

B, Q, N, E, D, H, ED = 16, 64, 32, 64, 128, 4, 8
G = B * Q
HD = H * D

GB = 128


def _bdot16(a, b, out_dtype=jnp.float32):
    return jax.lax.dot_general(
        a.astype(jnp.bfloat16), b.astype(jnp.bfloat16),
        dimension_numbers=(((2,), (1,)), ((0,), (0,))),
        preferred_element_type=out_dtype)


def _dot16(a, b, out_dtype=jnp.float32):
    return jnp.dot(a.astype(jnp.bfloat16), b.astype(jnp.bfloat16),
                   preferred_element_type=out_dtype)


def _gat_kernel(x_ref, src_ref, dst_ref, w_ref, e3_ref,
                ln1w_ref, ln1b_ref, wl_ref, bl_ref, wr_ref, br_ref,
                attbd_ref, onesbd_ref,
                gbias_ref, w1_ref, b1_ref, w2_ref, b2_ref,
                nw_ref, nb_ref, out_ref, xls_ref):
    bf16 = jnp.bfloat16

    x = x_ref[...]
    mu = jnp.mean(x, axis=-1, keepdims=True)
    d = (x - mu).astype(bf16)
    var = jnp.mean(d * d, axis=-1, keepdims=True)
    rs = jax.lax.rsqrt(var.astype(jnp.float32) + 1e-5).astype(bf16)
    xn = (d * rs) * ln1w_ref[...].astype(bf16) + ln1b_ref[...].astype(bf16)

    xl = (_dot16(xn, wl_ref[...]) + bl_ref[...]).astype(bf16)
    xr = (_dot16(xn, wr_ref[...]) + br_ref[...]).astype(bf16)
    xl_g = xl.reshape(GB, N, HD)
    xr_g = xr.reshape(GB, N, HD)

    src = src_ref[...]
    dst = dst_ref[...]
    srcb = src[:, :, None]
    dstb = dst[:, :, None]
    wb = w_ref[...][:, :, None]
    iota_n = jax.lax.broadcasted_iota(jnp.int32, (GB, E, N), 2)
    oh_dst = (dstb == iota_n).astype(bf16)
    iota_sl = jax.lax.broadcasted_iota(jnp.int32, (GB, N, E), 1)
    oh_dst_t = (dst[:, None, :] == iota_sl).astype(bf16)
    l72 = jax.lax.broadcasted_iota(jnp.int32, (GB, E, 2 * N + 8), 2)
    tgt = jnp.where(l72 < N, srcb,
                    jnp.where(l72 < 2 * N, dstb + N, wb + 2 * N))
    lhs_cat = (l72 == tgt).astype(bf16)
    l128 = jax.lax.broadcasted_iota(jnp.int32, (GB, E, H * N), 2)
    oh_src4 = ((l128 & (N - 1)) == srcb).astype(bf16)

    e3 = e3_ref[...].astype(bf16)
    att_bd = attbd_ref[...].astype(bf16)
    ones_bd32 = onesbd_ref[...].astype(bf16)

    e3_b = jnp.broadcast_to(e3[None], (GB, 8, HD))
    rhs_cat = jnp.concatenate([xl_g, xr_g, e3_b], axis=1)
    m = _bdot16(lhs_cat, rhs_cat).astype(bf16).reshape(GB * E, HD)

    m = jnp.where(m > 0, m, 0.2 * m)
    logits_e = jnp.dot(m, att_bd,
                       preferred_element_type=jnp.float32).reshape(GB, E, 8)

    ms = (xl + xr) + e3[2][None, :]
    ms = jnp.where(ms > 0, ms, 0.2 * ms)
    logits_s = jnp.dot(ms, att_bd,
                       preferred_element_type=jnp.float32).reshape(GB, N, 8)

    mx_dst = _bdot16(oh_dst, logits_s)
    aexp_e = jnp.exp(logits_e - mx_dst)

    den = _bdot16(oh_dst_t, aexp_e) + 1.0
    den_dst = _bdot16(oh_dst, den)
    alpha_e = aexp_e / (den_dst + 1e-16)
    alpha_s = 1.0 / (den + 1e-16)

    alpha_lane = _dot16(alpha_e.reshape(GB * E, 8), ones_bd32).astype(bf16)
    rhs_a = alpha_lane.reshape(GB, E, H * N) * oh_src4
    a_cat = _bdot16(oh_dst_t, rhs_a)
    alpha_s_lane = _dot16(alpha_s.reshape(GB * N, 8), ones_bd32)
    r_i = jax.lax.broadcasted_iota(jnp.int32, (N, H * N), 0)
    c_i = jax.lax.broadcasted_iota(jnp.int32, (N, H * N), 1)
    eye4 = (r_i == (c_i % N)).astype(jnp.float32)
    a_cat = (a_cat + eye4[None] * alpha_s_lane.reshape(GB, N, H * N)).astype(bf16)

    for h in range(H):
        xls_ref[:, h * N:(h + 1) * N, :] = xl_g[:, :, h * D:(h + 1) * D]
    out = _bdot16(a_cat, xls_ref[...])
    o = out.reshape(GB * N, D) + gbias_ref[...]

    h1 = _dot16(o, w1_ref[...]) + b1_ref[...]
    h1 = jnp.where(h1 > 0, h1, 0.01 * h1)
    h2 = _dot16(h1, w2_ref[...]) + b2_ref[...]

    gf = jnp.sum(h2.reshape(GB, N, D), axis=1) * (1.0 / N)
    mu2 = jnp.mean(gf, axis=-1, keepdims=True)
    var2 = jnp.mean((gf - mu2) ** 2, axis=-1, keepdims=True)
    out_ref[...] = (gf - mu2) * jax.lax.rsqrt(var2 + 1e-5) * nw_ref[...] + nb_ref[...]


@jax.jit
def _run(x_nodes, src, dst, ew, e3, ln1_w, ln1_b, WlT, bl, WrT, br,
         att_bd, ones_bd32, gat_bias, W1T, b1, W2T, b2, norm_w, norm_b):
    grid = (G // GB,)
    full = lambda shape: pl.BlockSpec(shape, lambda i: (0,) * len(shape))
    out = pl.pallas_call(
        _gat_kernel,
        grid=grid,
        in_specs=[
            pl.BlockSpec((GB * N, D), lambda i: (i, 0)),
            pl.BlockSpec((GB, E), lambda i: (i, 0)),
            pl.BlockSpec((GB, E), lambda i: (i, 0)),
            pl.BlockSpec((GB, E), lambda i: (i, 0)),
            full((8, HD)),
            full((1, D)), full((1, D)),
            full((D, HD)), full((1, HD)),
            full((D, HD)), full((1, HD)),
            full((HD, 8)),
            full((8, H * N)),
            full((1, D)),
            full((D, D)), full((1, D)),
            full((D, D)), full((1, D)),
            full((1, D)), full((1, D)),
        ],
        out_specs=pl.BlockSpec((GB, D), lambda i: (i, 0)),
        out_shape=jax.ShapeDtypeStruct((G, D), jnp.float32),
        scratch_shapes=[pltpu.VMEM((GB, H * N, D), jnp.bfloat16)],
    )(x_nodes, src, dst, ew, e3, ln1_w, ln1_b, WlT, bl, WrT, br,
      att_bd, ones_bd32, gat_bias, W1T, b1, W2T, b2, norm_w, norm_b)
    return out.reshape(B, Q, D)


def kernel(query_embeddings, query_maskings, edge_indexes, edge_weights,
           edge_maskings, edge_emb_table, ln1_w, ln1_b, Wl, bl, Wr, br, We,
           att, gat_bias, ffn_W1, ffn_b1, ffn_W2, ffn_b2, norm_w, norm_b):
    x_nodes = query_embeddings.reshape(G * N, D)
    eidx = edge_indexes.reshape(G, 2, E)
    src = eidx[:, 0, :]
    dst = eidx[:, 1, :]
    ew = edge_weights.reshape(G, E)

    p1 = jnp.mean(ew.astype(jnp.float32))
    ea_mean = edge_emb_table[0] + p1 * (edge_emb_table[1] - edge_emb_table[0])
    e3 = jnp.concatenate(
        [edge_emb_table, ea_mean[None, :],
         jnp.zeros((5, ED), jnp.float32)], axis=0) @ We.T

    hid = jnp.arange(HD, dtype=jnp.int32)
    h8 = jnp.arange(8, dtype=jnp.int32)
    att_bd = jnp.where((hid[:, None] // D) == h8[None, :],
                       att.reshape(-1)[:, None], 0.0)
    nid = jnp.arange(H * N, dtype=jnp.int32)
    ones_bd32 = ((h8[:, None] < H) & (nid[None, :] // N == h8[:, None])
                 ).astype(jnp.float32) * (1.0 / H)

    return _run(
        x_nodes, src, dst, ew, e3,
        ln1_w.reshape(1, D), ln1_b.reshape(1, D),
        Wl.T, bl.reshape(1, HD), Wr.T, br.reshape(1, HD),
        att_bd, ones_bd32, gat_bias.reshape(1, D),
        ffn_W1.T, ffn_b1.reshape(1, D), ffn_W2.T, ffn_b2.reshape(1, D),
        norm_w.reshape(1, D), norm_b.reshape(1, D))
